# baseline (speedup 1.0000x reference)
import functools

import numpy as np

import jax
import jax.numpy as jnp
from jax import lax
from jax.experimental import pallas as pl
from jax.experimental.pallas import tpu as pltpu
from jax.experimental.pallas import tpu_sc as plsc

N = 50000
DIM = 128
SL = 32
NSL = 4
NPT = 51200
NC = 2
NS = 16
L = 16
CH = 64
SUP = 16
A_TILE = 32768
D_TILE = 16384
A_PAD = A_TILE * NS
D_PAD = D_TILE * NS
ROWS_PER_TILE = NPT // NS


def _precompute_tables(h_pad, wa_parts, wd_parts):
    bn = 512

    def body(h_ref, wa_ref, wd_ref, ta_ref, td_ref):
        hb = h_ref[...]
        for p in range(3):
            hp = jnp.dot(hb, wa_ref[p],
                         preferred_element_type=jnp.float32).astype(jnp.bfloat16)
            for s in range(NSL):
                ta_ref[s * 3 + p] = hp[:, s * SL:(s + 1) * SL]
        for p in range(4):
            hp = jnp.dot(hb, wd_ref[p],
                         preferred_element_type=jnp.float32).astype(jnp.bfloat16)
            for s in range(NSL):
                td_ref[s * 4 + p] = hp[:, s * SL:(s + 1) * SL]

    return pl.pallas_call(
        body,
        grid=(NPT // bn,),
        in_specs=[
            pl.BlockSpec((bn, DIM), lambda i: (i, 0)),
            pl.BlockSpec((3, DIM, DIM), lambda i: (0, 0, 0)),
            pl.BlockSpec((4, DIM, DIM), lambda i: (0, 0, 0)),
        ],
        out_specs=[
            pl.BlockSpec((3 * NSL, bn, SL), lambda i: (0, i, 0)),
            pl.BlockSpec((4 * NSL, bn, SL), lambda i: (0, i, 0)),
        ],
        out_shape=[
            jax.ShapeDtypeStruct((3 * NSL, NPT, SL), jnp.bfloat16),
            jax.ShapeDtypeStruct((4 * NSL, NPT, SL), jnp.bfloat16),
        ],
    )(h_pad, wa_parts, wd_parts)


_SC_MESH = plsc.VectorSubcoreMesh(
    core_axis_name="c", subcore_axis_name="s", num_cores=NC, num_subcores=NS)


@functools.partial(
    pl.kernel,
    out_type=[
        jax.ShapeDtypeStruct((NSL, NPT, SL), jnp.float32),
        jax.ShapeDtypeStruct((NSL, NPT, SL), jnp.float32),
    ],
    mesh=_SC_MESH,
    compiler_params=pltpu.CompilerParams(use_tc_tiling_on_sc=False,
                                         needs_layout_passes=False),
    scratch_types=[
        pltpu.VMEM_SHARED((NPT, SL), jnp.float32),
        pltpu.VMEM((SUP, CH), jnp.int32),
        pltpu.VMEM((SUP, CH), jnp.int32),
        pltpu.VMEM((SUP, CH), jnp.int32),
        pltpu.VMEM((SUP, CH), jnp.int32),
        pltpu.VMEM((SUP, CH), jnp.float32),
    ] + [
        pltpu.VMEM((CH, SL), jnp.bfloat16)
        for _ in range(16)
    ] + [
        pltpu.VMEM((CH, SL), jnp.float32),
        pltpu.VMEM((CH, SL), jnp.float32),
        pltpu.VMEM((SL,), jnp.float32),
        pltpu.VMEM((SL,), jnp.float32),
        pltpu.SemaphoreType.DMA,
        pltpu.SemaphoreType.DMA,
        pltpu.SemaphoreType.DMA,
        pltpu.SemaphoreType.DMA,
    ],
)
def _sc_messages(ta, td, i0a, i1a, i2a, ava, i0d, i1d, i2d, i3d, avd,
                 wla, ba, wld, bd, agga, aggd,
                 acc, ib0, ib1, ib2, ib3, avb,
                 r00, r01, r02, r03, r10, r11, r12, r13,
                 r20, r21, r22, r23, r30, r31, r32, r33,
                 msg, cb, wlb, bb, sg0, sg1, sg2, sg3):
    c = lax.axis_index("c")
    t = lax.axis_index("s")
    rows0 = t * ROWS_PER_TILE

    for graph in (0, 1):
        if graph == 0:
            nparts, nsup = 3, A_TILE // (CH * SUP)
            idx_arrs = (i0a, i1a, i2a)
            avsrc, tab, wl_src, b_src, out = ava, ta, wla, ba, agga
        else:
            nparts, nsup = 4, D_TILE // (CH * SUP)
            idx_arrs = (i0d, i1d, i2d, i3d)
            avsrc, tab, wl_src, b_src, out = avd, td, wld, bd, aggd
        ibufs = (ib0, ib1, ib2, ib3)[:nparts]
        rslots = ((r00, r01, r02, r03)[:nparts], (r10, r11, r12, r13)[:nparts],
                  (r20, r21, r22, r23)[:nparts], (r30, r31, r32, r33)[:nparts])
        gsems = (sg0, sg1, sg2, sg3)
        tile_row0 = t * nsup * SUP

        def round_body(rr, _, nparts=nparts, nsup=nsup, idx_arrs=idx_arrs,
                       avsrc=avsrc, tab=tab, wl_src=wl_src, b_src=b_src,
                       out=out, ibufs=ibufs, tile_row0=tile_row0):
            s_dyn = 2 * rr + c

            def zrow(e, _):
                msg[e, pl.ds(0, L)] = jnp.zeros((L,), jnp.float32)
                msg[e, pl.ds(L, L)] = jnp.zeros((L,), jnp.float32)
                return 0
            lax.fori_loop(0, CH, zrow, 0)

            def zcopy(k, _):
                pltpu.sync_copy(msg, acc.at[pl.ds(rows0 + k * CH, CH)])
                return 0
            lax.fori_loop(0, ROWS_PER_TILE // CH, zcopy, 0)
            plsc.subcore_barrier()

            pltpu.sync_copy(wl_src.at[pl.ds(s_dyn * SL, SL)], wlb)
            pltpu.sync_copy(b_src.at[pl.ds(s_dyn * SL, SL)], bb)
            wl0 = wlb[pl.ds(0, L)]
            wl1 = wlb[pl.ds(L, L)]
            b0 = bb[pl.ds(0, L)]
            b1 = bb[pl.ds(L, L)]

            tabs = tuple(tab.at[s_dyn * nparts + p] for p in range(nparts))

            def fire(jj, s):
                for p in range(nparts):
                    pltpu.async_copy(tabs[p].at[ibufs[p].at[jj]],
                                     rslots[s][p], gsems[s])

            def wait_gathers(s):
                for p in range(nparts):
                    pltpu.make_async_copy(tabs[p].at[ibufs[p].at[0]],
                                          rslots[s][p], gsems[s]).wait()

            def compute(jj, s):
                rbufs = rslots[s]

                def stage(g, _):
                    av_v = avb[jj, pl.ds(g * L, L)]
                    for i in range(L):
                        e = g * L + i
                        av = av_v[i]
                        cb[e, pl.ds(0, L)] = av * wl0 + b0
                        cb[e, pl.ds(L, L)] = av * wl1 + b1
                    return 0
                lax.fori_loop(0, CH // L, stage, 0)

                def edge(e, _):
                    lo, hi = plsc.unpack(rbufs[0][e],
                                         format=plsc.PackFormat.INTERLEAVED,
                                         preferred_element_type=jnp.float32)
                    x0 = lo + cb[e, pl.ds(0, L)]
                    x1 = hi + cb[e, pl.ds(L, L)]
                    for p in range(1, nparts):
                        lo, hi = plsc.unpack(rbufs[p][e],
                                             format=plsc.PackFormat.INTERLEAVED,
                                             preferred_element_type=jnp.float32)
                        x0 = x0 + lo
                        x1 = x1 + hi
                    msg[e, pl.ds(0, L)] = x0 / (1.0 + jnp.exp(-x0))
                    msg[e, pl.ds(L, L)] = x1 / (1.0 + jnp.exp(-x1))
                    return 0
                lax.fori_loop(0, CH, edge, 0)
                pltpu.sync_copy(msg, acc.at[ibufs[1].at[jj]], add=True)

            def super_body(js, _):
                row = tile_row0 + js * SUP
                for p in range(nparts):
                    pltpu.sync_copy(idx_arrs[p].at[pl.ds(row, SUP)], ibufs[p])
                pltpu.sync_copy(avsrc.at[pl.ds(row, SUP)], avb)

                fire(0, 0)
                fire(1, 1)
                fire(2, 2)

                def qbody(q, _):
                    k0 = 4 * q
                    for m in range(4):
                        wait_gathers(m)
                        compute(k0 + m, m)
                        fire(k0 + m + 3, (m + 3) % 4)
                    return 0
                lax.fori_loop(0, SUP // 4 - 1, qbody, 0)

                fire(SUP - 1, 3)
                for m in range(4):
                    wait_gathers(m)
                    compute(SUP - 4 + m, m)
                return 0
            lax.fori_loop(0, nsup, super_body, 0)

            plsc.subcore_barrier()
            pltpu.sync_copy(acc.at[pl.ds(rows0, ROWS_PER_TILE)],
                            out.at[s_dyn, pl.ds(rows0, ROWS_PER_TILE)])
            plsc.subcore_barrier()
            return 0
        lax.fori_loop(0, 2, round_body, 0)


def _node_update(h, agga, aggd, whh, wha, whd, bh):
    bn = 400

    def body(h_ref, a_ref, d_ref, whh_ref, wha_ref, whd_ref, bh_ref, o_ref):
        hb = h_ref[...]
        acc = jnp.dot(hb, whh_ref[...], preferred_element_type=jnp.float32)
        for s in range(NSL):
            acc = acc + jnp.dot(a_ref[s], wha_ref[s],
                                preferred_element_type=jnp.float32)
            acc = acc + jnp.dot(d_ref[s], whd_ref[s],
                                preferred_element_type=jnp.float32)
        o_ref[...] = hb + acc + bh_ref[...]

    return pl.pallas_call(
        body,
        grid=(N // bn,),
        in_specs=[
            pl.BlockSpec((bn, DIM), lambda i: (i, 0)),
            pl.BlockSpec((NSL, bn, SL), lambda i: (0, i, 0)),
            pl.BlockSpec((NSL, bn, SL), lambda i: (0, i, 0)),
            pl.BlockSpec((DIM, DIM), lambda i: (0, 0)),
            pl.BlockSpec((NSL, SL, DIM), lambda i: (0, 0, 0)),
            pl.BlockSpec((NSL, SL, DIM), lambda i: (0, 0, 0)),
            pl.BlockSpec((1, DIM), lambda i: (0, 0)),
        ],
        out_specs=pl.BlockSpec((bn, DIM), lambda i: (i, 0)),
        out_shape=jax.ShapeDtypeStruct((N, DIM), jnp.float32),
    )(h, agga, aggd, whh, wha, whd, bh)


def _pad_edges(idx, val, e_pad):
    e = idx.shape[0]
    pad = e_pad - e
    cols = tuple(
        jnp.concatenate([idx[:, k], jnp.full((pad,), N, jnp.int32)])
        .reshape(-1, CH)
        for k in range(idx.shape[1]))
    v = jnp.concatenate([val[:, 0], jnp.zeros((pad,), jnp.float32)]).reshape(-1, CH)
    return cols, v


def kernel(h, a_idx, a_val, d_idx, d_val, W_a, b_a, W_d, b_d, W_h, b_h):
    perm = np.concatenate([np.arange(0, SL, 2), np.arange(1, SL, 2)])
    wa_parts = W_a[:3 * DIM].reshape(3, DIM, DIM)
    wla = W_a[3 * DIM].reshape(NSL, SL)[:, perm].reshape(DIM)
    ba = b_a.reshape(NSL, SL)[:, perm].reshape(DIM)
    wd_parts = W_d[:4 * DIM].reshape(4, DIM, DIM)
    wld = W_d[4 * DIM].reshape(NSL, SL)[:, perm].reshape(DIM)
    bd = b_d.reshape(NSL, SL)[:, perm].reshape(DIM)
    whh = W_h[:DIM]
    wha = W_h[DIM:2 * DIM].reshape(NSL, SL, DIM)[:, perm, :]
    whd = W_h[2 * DIM:3 * DIM].reshape(NSL, SL, DIM)[:, perm, :]
    bh = b_h.reshape(1, DIM)

    h_pad = jnp.pad(h, ((0, NPT - N), (0, 0)))
    ta, td = _precompute_tables(h_pad, wa_parts, wd_parts)

    (i0a, i1a, i2a), ava = _pad_edges(a_idx, a_val, A_PAD)
    (i0d, i1d, i2d, i3d), avd = _pad_edges(d_idx, d_val, D_PAD)

    agga, aggd = _sc_messages(ta, td, i0a, i1a, i2a, ava,
                              i0d, i1d, i2d, i3d, avd, wla, ba, wld, bd)

    return _node_update(h, agga, aggd, whh, wha, whd, bh)


# speedup vs baseline: 1.2691x; 1.2691x over previous
import functools

import jax
import jax.numpy as jnp
from jax import lax
from jax.experimental import pallas as pl
from jax.experimental.pallas import tpu as pltpu
from jax.experimental.pallas import tpu_sc as plsc

N = 50000
DIM = 128
SL = 32
NSL = 4
NPT = 51200
NC = 2
NS = 16
L = 16
CH = 64
SUP = 16
A_TILE = 32768
D_TILE = 16384
A_PAD = A_TILE * NS
D_PAD = D_TILE * NS
ROWS_PER_TILE = NPT // NS


def _precompute_tables(h_pad, wa_parts, wd_parts):
    bn = 512

    def body(h_ref, wa_ref, wd_ref, ta_ref, td_ref):
        hb = h_ref[...]
        for p in range(3):
            hp = jnp.dot(hb, wa_ref[p], preferred_element_type=jnp.float32)
            for s in range(NSL):
                ta_ref[s * 3 + p] = hp[:, s * SL:(s + 1) * SL]
        for p in range(4):
            hp = jnp.dot(hb, wd_ref[p], preferred_element_type=jnp.float32)
            for s in range(NSL):
                td_ref[s * 4 + p] = hp[:, s * SL:(s + 1) * SL]

    return pl.pallas_call(
        body,
        grid=(NPT // bn,),
        in_specs=[
            pl.BlockSpec((bn, DIM), lambda i: (i, 0)),
            pl.BlockSpec((3, DIM, DIM), lambda i: (0, 0, 0)),
            pl.BlockSpec((4, DIM, DIM), lambda i: (0, 0, 0)),
        ],
        out_specs=[
            pl.BlockSpec((3 * NSL, bn, SL), lambda i: (0, i, 0)),
            pl.BlockSpec((4 * NSL, bn, SL), lambda i: (0, i, 0)),
        ],
        out_shape=[
            jax.ShapeDtypeStruct((3 * NSL, NPT, SL), jnp.float32),
            jax.ShapeDtypeStruct((4 * NSL, NPT, SL), jnp.float32),
        ],
    )(h_pad, wa_parts, wd_parts)


_SC_MESH = plsc.VectorSubcoreMesh(
    core_axis_name="c", subcore_axis_name="s", num_cores=NC, num_subcores=NS)


@functools.partial(
    pl.kernel,
    out_type=[
        jax.ShapeDtypeStruct((NSL, NPT, SL), jnp.float32),
        jax.ShapeDtypeStruct((NSL, NPT, SL), jnp.float32),
    ],
    mesh=_SC_MESH,
    compiler_params=pltpu.CompilerParams(use_tc_tiling_on_sc=False),
    scratch_types=[
        pltpu.VMEM_SHARED((NPT, SL), jnp.float32),
        pltpu.VMEM((SUP, CH), jnp.int32),
        pltpu.VMEM((SUP, CH), jnp.int32),
        pltpu.VMEM((SUP, CH), jnp.int32),
        pltpu.VMEM((SUP, CH), jnp.int32),
        pltpu.VMEM((SUP, CH), jnp.float32),
    ] + [
        pltpu.VMEM((CH, SL), jnp.float32)
        for _ in range(8)
    ] + [
        pltpu.VMEM((CH, SL), jnp.float32),
        pltpu.VMEM((CH, SL), jnp.float32),
        pltpu.VMEM((CH, SL), jnp.float32),
        pltpu.VMEM((SL,), jnp.float32),
        pltpu.VMEM((SL,), jnp.float32),
        pltpu.SemaphoreType.DMA,
        pltpu.SemaphoreType.DMA,
    ],
)
def _sc_messages(ta, td, i0a, i1a, i2a, ava, i0d, i1d, i2d, i3d, avd,
                 wla, ba, wld, bd, agga, aggd,
                 acc, ib0, ib1, ib2, ib3, avb,
                 ra0, ra1, ra2, ra3, rb0, rb1, rb2, rb3,
                 msga, msgb, cb, wlb, bb, sg0, sg1):
    c = lax.axis_index("c")
    t = lax.axis_index("s")
    rows0 = t * ROWS_PER_TILE

    for graph in (0, 1):
        if graph == 0:
            nparts, nsup = 3, A_TILE // (CH * SUP)
            idx_arrs = (i0a, i1a, i2a)
            avsrc, tab, wl_src, b_src, out = ava, ta, wla, ba, agga
        else:
            nparts, nsup = 4, D_TILE // (CH * SUP)
            idx_arrs = (i0d, i1d, i2d, i3d)
            avsrc, tab, wl_src, b_src, out = avd, td, wld, bd, aggd
        ibufs = (ib0, ib1, ib2, ib3)[:nparts]
        rslots = ((ra0, ra1, ra2, ra3)[:nparts], (rb0, rb1, rb2, rb3)[:nparts])
        mslots = (msga, msgb)
        gsems = (sg0, sg1)
        tile_row0 = t * nsup * SUP

        def round_body(rr, _, nparts=nparts, nsup=nsup, idx_arrs=idx_arrs,
                       avsrc=avsrc, tab=tab, wl_src=wl_src, b_src=b_src,
                       out=out, ibufs=ibufs, tile_row0=tile_row0):
            s_dyn = 2 * rr + c

            def zrow(e, _):
                msga[e, pl.ds(0, L)] = jnp.zeros((L,), jnp.float32)
                msga[e, pl.ds(L, L)] = jnp.zeros((L,), jnp.float32)
                return 0
            lax.fori_loop(0, CH, zrow, 0)

            def zcopy(k, _):
                pltpu.sync_copy(msga, acc.at[pl.ds(rows0 + k * CH, CH)])
                return 0
            lax.fori_loop(0, ROWS_PER_TILE // CH, zcopy, 0)
            plsc.subcore_barrier()

            pltpu.sync_copy(wl_src.at[pl.ds(s_dyn * SL, SL)], wlb)
            pltpu.sync_copy(b_src.at[pl.ds(s_dyn * SL, SL)], bb)
            wl0 = wlb[pl.ds(0, L)]
            wl1 = wlb[pl.ds(L, L)]
            b0 = bb[pl.ds(0, L)]
            b1 = bb[pl.ds(L, L)]

            tabs = tuple(tab.at[s_dyn * nparts + p] for p in range(nparts))

            def fire(jj, s):
                for p in range(nparts):
                    pltpu.async_copy(tabs[p].at[ibufs[p].at[jj]],
                                     rslots[s][p], gsems[s])

            def wait_gathers(s):
                for p in range(nparts):
                    pltpu.make_async_copy(tabs[p].at[ibufs[p].at[0]],
                                          rslots[s][p], gsems[s]).wait()

            def compute(jj, s):
                rbufs = rslots[s]
                msg = mslots[s]

                def stage(g, _):
                    av_v = avb[jj, pl.ds(g * L, L)]
                    for i in range(L):
                        e = g * L + i
                        av = av_v[i]
                        cb[e, pl.ds(0, L)] = av * wl0 + b0
                        cb[e, pl.ds(L, L)] = av * wl1 + b1
                    return 0
                lax.fori_loop(0, CH // L, stage, 0)

                def edge(e, _):
                    x0 = rbufs[0][e, pl.ds(0, L)] + cb[e, pl.ds(0, L)]
                    x1 = rbufs[0][e, pl.ds(L, L)] + cb[e, pl.ds(L, L)]
                    for p in range(1, nparts):
                        x0 = x0 + rbufs[p][e, pl.ds(0, L)]
                        x1 = x1 + rbufs[p][e, pl.ds(L, L)]
                    msg[e, pl.ds(0, L)] = x0 / (1.0 + jnp.exp(-x0))
                    msg[e, pl.ds(L, L)] = x1 / (1.0 + jnp.exp(-x1))
                    return 0
                lax.fori_loop(0, CH, edge, 0)
                pltpu.sync_copy(msg, acc.at[ibufs[1].at[jj]], add=True)

            def super_body(js, _):
                row = tile_row0 + js * SUP
                for p in range(nparts):
                    pltpu.sync_copy(idx_arrs[p].at[pl.ds(row, SUP)], ibufs[p])
                pltpu.sync_copy(avsrc.at[pl.ds(row, SUP)], avb)

                fire(0, 0)

                def qbody(q, _):
                    k0 = 2 * q
                    fire(k0 + 1, 1)
                    wait_gathers(0)
                    compute(k0, 0)
                    fire(k0 + 2, 0)
                    wait_gathers(1)
                    compute(k0 + 1, 1)
                    return 0
                lax.fori_loop(0, SUP // 2 - 1, qbody, 0)

                fire(SUP - 1, 1)
                wait_gathers(0)
                compute(SUP - 2, 0)
                wait_gathers(1)
                compute(SUP - 1, 1)
                return 0
            lax.fori_loop(0, nsup, super_body, 0)

            plsc.subcore_barrier()
            pltpu.sync_copy(acc.at[pl.ds(rows0, ROWS_PER_TILE)],
                            out.at[s_dyn, pl.ds(rows0, ROWS_PER_TILE)])
            plsc.subcore_barrier()
            return 0
        lax.fori_loop(0, 2, round_body, 0)


def _node_update(h, agga, aggd, whh, wha, whd, bh):
    bn = 400

    def body(h_ref, a_ref, d_ref, whh_ref, wha_ref, whd_ref, bh_ref, o_ref):
        hb = h_ref[...]
        acc = jnp.dot(hb, whh_ref[...], preferred_element_type=jnp.float32)
        for s in range(NSL):
            acc = acc + jnp.dot(a_ref[s], wha_ref[s],
                                preferred_element_type=jnp.float32)
            acc = acc + jnp.dot(d_ref[s], whd_ref[s],
                                preferred_element_type=jnp.float32)
        o_ref[...] = hb + acc + bh_ref[...]

    return pl.pallas_call(
        body,
        grid=(N // bn,),
        in_specs=[
            pl.BlockSpec((bn, DIM), lambda i: (i, 0)),
            pl.BlockSpec((NSL, bn, SL), lambda i: (0, i, 0)),
            pl.BlockSpec((NSL, bn, SL), lambda i: (0, i, 0)),
            pl.BlockSpec((DIM, DIM), lambda i: (0, 0)),
            pl.BlockSpec((NSL, SL, DIM), lambda i: (0, 0, 0)),
            pl.BlockSpec((NSL, SL, DIM), lambda i: (0, 0, 0)),
            pl.BlockSpec((1, DIM), lambda i: (0, 0)),
        ],
        out_specs=pl.BlockSpec((bn, DIM), lambda i: (i, 0)),
        out_shape=jax.ShapeDtypeStruct((N, DIM), jnp.float32),
    )(h, agga, aggd, whh, wha, whd, bh)


def _pad_edges(idx, val, e_pad):
    e = idx.shape[0]
    pad = e_pad - e
    cols = tuple(
        jnp.concatenate([idx[:, k], jnp.full((pad,), N, jnp.int32)])
        .reshape(-1, CH)
        for k in range(idx.shape[1]))
    v = jnp.concatenate([val[:, 0], jnp.zeros((pad,), jnp.float32)]).reshape(-1, CH)
    return cols, v


def kernel(h, a_idx, a_val, d_idx, d_val, W_a, b_a, W_d, b_d, W_h, b_h):
    wa_parts = W_a[:3 * DIM].reshape(3, DIM, DIM)
    wla = W_a[3 * DIM]
    ba = b_a
    wd_parts = W_d[:4 * DIM].reshape(4, DIM, DIM)
    wld = W_d[4 * DIM]
    bd = b_d
    whh = W_h[:DIM]
    wha = W_h[DIM:2 * DIM].reshape(NSL, SL, DIM)
    whd = W_h[2 * DIM:3 * DIM].reshape(NSL, SL, DIM)
    bh = b_h.reshape(1, DIM)

    h_pad = jnp.pad(h, ((0, NPT - N), (0, 0)))
    ta, td = _precompute_tables(h_pad, wa_parts, wd_parts)

    (i0a, i1a, i2a), ava = _pad_edges(a_idx, a_val, A_PAD)
    (i0d, i1d, i2d, i3d), avd = _pad_edges(d_idx, d_val, D_PAD)

    agga, aggd = _sc_messages(ta, td, i0a, i1a, i2a, ava,
                              i0d, i1d, i2d, i3d, avd, wla, ba, wld, bd)

    return _node_update(h, agga, aggd, whh, wha, whd, bh)


# speedup vs baseline: 1.3210x; 1.0409x over previous
import functools

import jax
import jax.numpy as jnp
from jax import lax
from jax.experimental import pallas as pl
from jax.experimental.pallas import tpu as pltpu
from jax.experimental.pallas import tpu_sc as plsc

N = 50000
DIM = 128
SL = 32
NSL = 4
NPT = 51200
NC = 2
NS = 16
L = 16
CH = 64
SUP = 16
A_TILE = 32768
D_TILE = 16384
A_PAD = A_TILE * NS
D_PAD = D_TILE * NS
ROWS_PER_TILE = NPT // NS


def _precompute_tables(h_pad, wa_parts, wd_parts):
    bn = 512

    def body(h_ref, wa_ref, wd_ref, ta_ref, td_ref):
        hb = h_ref[...]
        for p in range(3):
            hp = jnp.dot(hb, wa_ref[p], preferred_element_type=jnp.float32)
            for s in range(NSL):
                ta_ref[s * 3 + p] = hp[:, s * SL:(s + 1) * SL]
        for p in range(4):
            hp = jnp.dot(hb, wd_ref[p], preferred_element_type=jnp.float32)
            for s in range(NSL):
                td_ref[s * 4 + p] = hp[:, s * SL:(s + 1) * SL]

    return pl.pallas_call(
        body,
        grid=(NPT // bn,),
        in_specs=[
            pl.BlockSpec((bn, DIM), lambda i: (i, 0)),
            pl.BlockSpec((3, DIM, DIM), lambda i: (0, 0, 0)),
            pl.BlockSpec((4, DIM, DIM), lambda i: (0, 0, 0)),
        ],
        out_specs=[
            pl.BlockSpec((3 * NSL, bn, SL), lambda i: (0, i, 0)),
            pl.BlockSpec((4 * NSL, bn, SL), lambda i: (0, i, 0)),
        ],
        out_shape=[
            jax.ShapeDtypeStruct((3 * NSL, NPT, SL), jnp.float32),
            jax.ShapeDtypeStruct((4 * NSL, NPT, SL), jnp.float32),
        ],
    )(h_pad, wa_parts, wd_parts)


_SC_MESH = plsc.VectorSubcoreMesh(
    core_axis_name="c", subcore_axis_name="s", num_cores=NC, num_subcores=NS)


@functools.partial(
    pl.kernel,
    out_type=[
        jax.ShapeDtypeStruct((NSL, NPT, SL), jnp.float32),
        jax.ShapeDtypeStruct((NSL, NPT, SL), jnp.float32),
    ],
    mesh=_SC_MESH,
    compiler_params=pltpu.CompilerParams(use_tc_tiling_on_sc=False),
    scratch_types=[
        pltpu.VMEM_SHARED((NPT, SL), jnp.float32),
        pltpu.VMEM((SUP, CH), jnp.int32),
        pltpu.VMEM((SUP, CH), jnp.int32),
        pltpu.VMEM((SUP, CH), jnp.int32),
        pltpu.VMEM((SUP, CH), jnp.int32),
        pltpu.VMEM((SUP, CH), jnp.float32),
    ] + [
        pltpu.VMEM((CH, SL), jnp.float32)
        for _ in range(8)
    ] + [
        pltpu.VMEM((CH, SL), jnp.float32),
        pltpu.VMEM((CH, SL), jnp.float32),
        pltpu.VMEM((CH, SL), jnp.float32),
        pltpu.VMEM((SL,), jnp.float32),
        pltpu.VMEM((SL,), jnp.float32),
        pltpu.SemaphoreType.DMA,
        pltpu.SemaphoreType.DMA,
        pltpu.SemaphoreType.DMA,
    ],
)
def _sc_messages(ta, td, i0a, i1a, i2a, ava, i0d, i1d, i2d, i3d, avd,
                 wla, ba, wld, bd, agga, aggd,
                 acc, ib0, ib1, ib2, ib3, avb,
                 ra0, ra1, ra2, ra3, rb0, rb1, rb2, rb3,
                 msga, msgb, cb, wlb, bb, sg0, sg1, si):
    c = lax.axis_index("c")
    t = lax.axis_index("s")
    rows0 = t * ROWS_PER_TILE

    for graph in (0, 1):
        if graph == 0:
            nparts, nsup = 3, A_TILE // (CH * SUP)
            idx_arrs = (i0a, i1a, i2a)
            avsrc, tab, wl_src, b_src, out = ava, ta, wla, ba, agga
        else:
            nparts, nsup = 4, D_TILE // (CH * SUP)
            idx_arrs = (i0d, i1d, i2d, i3d)
            avsrc, tab, wl_src, b_src, out = avd, td, wld, bd, aggd
        ibufs = (ib0, ib1, ib2, ib3)[:nparts]
        rslots = ((ra0, ra1, ra2, ra3)[:nparts], (rb0, rb1, rb2, rb3)[:nparts])
        mslots = (msga, msgb)
        gsems = (sg0, sg1)
        tile_row0 = t * nsup * SUP

        def round_body(rr, _, nparts=nparts, nsup=nsup, idx_arrs=idx_arrs,
                       avsrc=avsrc, tab=tab, wl_src=wl_src, b_src=b_src,
                       out=out, ibufs=ibufs, tile_row0=tile_row0):
            s_dyn = 2 * rr + c

            def zrow(e, _):
                msga[e, pl.ds(0, L)] = jnp.zeros((L,), jnp.float32)
                msga[e, pl.ds(L, L)] = jnp.zeros((L,), jnp.float32)
                return 0
            lax.fori_loop(0, CH, zrow, 0)

            def zcopy(k, _):
                pltpu.sync_copy(msga, acc.at[pl.ds(rows0 + k * CH, CH)])
                return 0
            lax.fori_loop(0, ROWS_PER_TILE // CH, zcopy, 0)
            plsc.subcore_barrier()

            pltpu.sync_copy(wl_src.at[pl.ds(s_dyn * SL, SL)], wlb)
            pltpu.sync_copy(b_src.at[pl.ds(s_dyn * SL, SL)], bb)
            wl0 = wlb[pl.ds(0, L)]
            wl1 = wlb[pl.ds(L, L)]
            b0 = bb[pl.ds(0, L)]
            b1 = bb[pl.ds(L, L)]

            tabs = tuple(tab.at[s_dyn * nparts + p] for p in range(nparts))

            def fire(jj, s):
                for p in range(nparts):
                    pltpu.async_copy(tabs[p].at[ibufs[p].at[jj]],
                                     rslots[s][p], gsems[s])

            def wait_gathers(s):
                for p in range(nparts):
                    pltpu.make_async_copy(tabs[p].at[ibufs[p].at[0]],
                                          rslots[s][p], gsems[s]).wait()

            def compute(jj, s):
                rbufs = rslots[s]
                msg = mslots[s]

                def stage(g, _):
                    av_v = avb[jj, pl.ds(g * L, L)]
                    for i in range(L):
                        e = g * L + i
                        av = av_v[i]
                        cb[e, pl.ds(0, L)] = av * wl0 + b0
                        cb[e, pl.ds(L, L)] = av * wl1 + b1
                    return 0
                lax.fori_loop(0, CH // L, stage, 0)

                def edge(e, _):
                    x0 = rbufs[0][e, pl.ds(0, L)] + cb[e, pl.ds(0, L)]
                    x1 = rbufs[0][e, pl.ds(L, L)] + cb[e, pl.ds(L, L)]
                    for p in range(1, nparts):
                        x0 = x0 + rbufs[p][e, pl.ds(0, L)]
                        x1 = x1 + rbufs[p][e, pl.ds(L, L)]
                    msg[e, pl.ds(0, L)] = x0 / (1.0 + jnp.exp(-x0))
                    msg[e, pl.ds(L, L)] = x1 / (1.0 + jnp.exp(-x1))
                    return 0
                lax.fori_loop(0, CH, edge, 0)
                pltpu.sync_copy(msg, acc.at[ibufs[1].at[jj]], add=True)

            def super_body(js, _):
                row = tile_row0 + js * SUP
                descs = [
                    pltpu.async_copy(idx_arrs[p].at[pl.ds(row, SUP)],
                                     ibufs[p], si)
                    for p in range(nparts)
                ]
                descs.append(pltpu.async_copy(avsrc.at[pl.ds(row, SUP)],
                                              avb, si))
                for d in descs:
                    d.wait()

                fire(0, 0)

                def qbody(q, _):
                    k0 = 2 * q
                    fire(k0 + 1, 1)
                    wait_gathers(0)
                    compute(k0, 0)
                    fire(k0 + 2, 0)
                    wait_gathers(1)
                    compute(k0 + 1, 1)
                    return 0
                lax.fori_loop(0, SUP // 2 - 1, qbody, 0)

                fire(SUP - 1, 1)
                wait_gathers(0)
                compute(SUP - 2, 0)
                wait_gathers(1)
                compute(SUP - 1, 1)
                return 0
            lax.fori_loop(0, nsup, super_body, 0)

            plsc.subcore_barrier()
            pltpu.sync_copy(acc.at[pl.ds(rows0, ROWS_PER_TILE)],
                            out.at[s_dyn, pl.ds(rows0, ROWS_PER_TILE)])
            plsc.subcore_barrier()
            return 0
        lax.fori_loop(0, 2, round_body, 0)


def _node_update(h, agga, aggd, whh, wha, whd, bh):
    bn = 400

    def body(h_ref, a_ref, d_ref, whh_ref, wha_ref, whd_ref, bh_ref, o_ref):
        hb = h_ref[...]
        acc = jnp.dot(hb, whh_ref[...], preferred_element_type=jnp.float32)
        for s in range(NSL):
            acc = acc + jnp.dot(a_ref[s], wha_ref[s],
                                preferred_element_type=jnp.float32)
            acc = acc + jnp.dot(d_ref[s], whd_ref[s],
                                preferred_element_type=jnp.float32)
        o_ref[...] = hb + acc + bh_ref[...]

    return pl.pallas_call(
        body,
        grid=(N // bn,),
        in_specs=[
            pl.BlockSpec((bn, DIM), lambda i: (i, 0)),
            pl.BlockSpec((NSL, bn, SL), lambda i: (0, i, 0)),
            pl.BlockSpec((NSL, bn, SL), lambda i: (0, i, 0)),
            pl.BlockSpec((DIM, DIM), lambda i: (0, 0)),
            pl.BlockSpec((NSL, SL, DIM), lambda i: (0, 0, 0)),
            pl.BlockSpec((NSL, SL, DIM), lambda i: (0, 0, 0)),
            pl.BlockSpec((1, DIM), lambda i: (0, 0)),
        ],
        out_specs=pl.BlockSpec((bn, DIM), lambda i: (i, 0)),
        out_shape=jax.ShapeDtypeStruct((N, DIM), jnp.float32),
    )(h, agga, aggd, whh, wha, whd, bh)


def _pad_edges(idx, val, e_pad):
    e = idx.shape[0]
    pad = e_pad - e
    cols = tuple(
        jnp.concatenate([idx[:, k], jnp.full((pad,), N, jnp.int32)])
        .reshape(-1, CH)
        for k in range(idx.shape[1]))
    v = jnp.concatenate([val[:, 0], jnp.zeros((pad,), jnp.float32)]).reshape(-1, CH)
    return cols, v


def kernel(h, a_idx, a_val, d_idx, d_val, W_a, b_a, W_d, b_d, W_h, b_h):
    wa_parts = W_a[:3 * DIM].reshape(3, DIM, DIM)
    wla = W_a[3 * DIM]
    ba = b_a
    wd_parts = W_d[:4 * DIM].reshape(4, DIM, DIM)
    wld = W_d[4 * DIM]
    bd = b_d
    whh = W_h[:DIM]
    wha = W_h[DIM:2 * DIM].reshape(NSL, SL, DIM)
    whd = W_h[2 * DIM:3 * DIM].reshape(NSL, SL, DIM)
    bh = b_h.reshape(1, DIM)

    h_pad = jnp.pad(h, ((0, NPT - N), (0, 0)))
    ta, td = _precompute_tables(h_pad, wa_parts, wd_parts)

    (i0a, i1a, i2a), ava = _pad_edges(a_idx, a_val, A_PAD)
    (i0d, i1d, i2d, i3d), avd = _pad_edges(d_idx, d_val, D_PAD)

    agga, aggd = _sc_messages(ta, td, i0a, i1a, i2a, ava,
                              i0d, i1d, i2d, i3d, avd, wla, ba, wld, bd)

    return _node_update(h, agga, aggd, whh, wha, whd, bh)


# speedup vs baseline: 1.6428x; 1.2436x over previous
import functools

import jax
import jax.numpy as jnp
from jax import lax
from jax.experimental import pallas as pl
from jax.experimental.pallas import tpu as pltpu
from jax.experimental.pallas import tpu_sc as plsc

N = 50000
DIM = 128
SL = 32
NSL = 4
NPT = 51200
NC = 2
NS = 16
L = 16
CH = 64
SUP = 12
A_TILE = 31488
D_TILE = 16128
A_PAD = A_TILE * NS
D_PAD = D_TILE * NS
ROWS_PER_TILE = NPT // NS


def _precompute_tables(h_pad, wa_parts, wd_parts):
    bn = 512

    def body(h_ref, wa_ref, wd_ref, ta_ref, td_ref):
        hb = h_ref[...]
        for p in range(3):
            hp = jnp.dot(hb, wa_ref[p], preferred_element_type=jnp.float32)
            for s in range(NSL):
                ta_ref[s * 3 + p] = hp[:, s * SL:(s + 1) * SL]
        for p in range(4):
            hp = jnp.dot(hb, wd_ref[p], preferred_element_type=jnp.float32)
            for s in range(NSL):
                td_ref[s * 4 + p] = hp[:, s * SL:(s + 1) * SL]

    return pl.pallas_call(
        body,
        grid=(NPT // bn,),
        in_specs=[
            pl.BlockSpec((bn, DIM), lambda i: (i, 0)),
            pl.BlockSpec((3, DIM, DIM), lambda i: (0, 0, 0)),
            pl.BlockSpec((4, DIM, DIM), lambda i: (0, 0, 0)),
        ],
        out_specs=[
            pl.BlockSpec((3 * NSL, bn, SL), lambda i: (0, i, 0)),
            pl.BlockSpec((4 * NSL, bn, SL), lambda i: (0, i, 0)),
        ],
        out_shape=[
            jax.ShapeDtypeStruct((3 * NSL, NPT, SL), jnp.float32),
            jax.ShapeDtypeStruct((4 * NSL, NPT, SL), jnp.float32),
        ],
    )(h_pad, wa_parts, wd_parts)


_SC_MESH = plsc.VectorSubcoreMesh(
    core_axis_name="c", subcore_axis_name="s", num_cores=NC, num_subcores=NS)


@functools.partial(
    pl.kernel,
    out_type=[
        jax.ShapeDtypeStruct((NSL, NPT, SL), jnp.float32),
        jax.ShapeDtypeStruct((NSL, NPT, SL), jnp.float32),
    ],
    mesh=_SC_MESH,
    compiler_params=pltpu.CompilerParams(use_tc_tiling_on_sc=False),
    scratch_types=[
        pltpu.VMEM_SHARED((NPT, SL), jnp.float32),
        pltpu.VMEM((SUP, CH), jnp.int32),
        pltpu.VMEM((SUP, CH), jnp.int32),
        pltpu.VMEM((SUP, CH), jnp.int32),
        pltpu.VMEM((SUP, CH), jnp.int32),
        pltpu.VMEM((SUP, CH), jnp.float32),
    ] + [
        pltpu.VMEM((CH, SL), jnp.float32)
        for _ in range(9)
    ] + [
        pltpu.VMEM((CH, SL), jnp.float32),
        pltpu.VMEM((CH, SL), jnp.float32),
        pltpu.VMEM((SL,), jnp.float32),
        pltpu.VMEM((SL,), jnp.float32),
        pltpu.SemaphoreType.DMA,
        pltpu.SemaphoreType.DMA,
        pltpu.SemaphoreType.DMA,
        pltpu.SemaphoreType.DMA,
    ],
)
def _sc_messages(ta, td, i0a, i1a, i2a, ava, i0d, i1d, i2d, i3d, avd,
                 wla, ba, wld, bd, agga, aggd,
                 acc, ib0, ib1, ib2, ib3, avb,
                 p0, p1, p2, p3, p4, p5, p6, p7, p8,
                 msg, cb, wlb, bb, sg0, sg1, sg2, si):
    c = lax.axis_index("c")
    t = lax.axis_index("s")
    rows0 = t * ROWS_PER_TILE

    for graph in (0, 1):
        if graph == 0:
            nparts, nsup = 3, A_TILE // (CH * SUP)
            idx_arrs = (i0a, i1a, i2a)
            avsrc, tab, wl_src, b_src, out = ava, ta, wla, ba, agga
        else:
            nparts, nsup = 4, D_TILE // (CH * SUP)
            idx_arrs = (i0d, i1d, i2d, i3d)
            avsrc, tab, wl_src, b_src, out = avd, td, wld, bd, aggd
        ibufs = (ib0, ib1, ib2, ib3)[:nparts]
        if graph == 0:
            nslots = 3
            rslots = ((p0, p1, p2), (p3, p4, p5), (p6, p7, p8))
        else:
            nslots = 2
            rslots = ((p0, p1, p2, p3), (p4, p5, p6, p7))
        gsems = (sg0, sg1, sg2)[:nslots]
        tile_row0 = t * nsup * SUP

        def round_body(rr, _, nparts=nparts, nsup=nsup, idx_arrs=idx_arrs,
                       avsrc=avsrc, tab=tab, wl_src=wl_src, b_src=b_src,
                       out=out, ibufs=ibufs, tile_row0=tile_row0):
            s_dyn = 2 * rr + c

            def zrow(e, _):
                msg[e, pl.ds(0, L)] = jnp.zeros((L,), jnp.float32)
                msg[e, pl.ds(L, L)] = jnp.zeros((L,), jnp.float32)
                return 0
            lax.fori_loop(0, CH, zrow, 0)

            def zcopy(k, _):
                pltpu.sync_copy(msg, acc.at[pl.ds(rows0 + k * CH, CH)])
                return 0
            lax.fori_loop(0, ROWS_PER_TILE // CH, zcopy, 0)
            plsc.subcore_barrier()

            pltpu.sync_copy(wl_src.at[pl.ds(s_dyn * SL, SL)], wlb)
            pltpu.sync_copy(b_src.at[pl.ds(s_dyn * SL, SL)], bb)
            wl0 = wlb[pl.ds(0, L)]
            wl1 = wlb[pl.ds(L, L)]
            b0 = bb[pl.ds(0, L)]
            b1 = bb[pl.ds(L, L)]

            tabs = tuple(tab.at[s_dyn * nparts + p] for p in range(nparts))

            def fire(jj, s):
                for p in range(nparts):
                    pltpu.async_copy(tabs[p].at[ibufs[p].at[jj]],
                                     rslots[s][p], gsems[s])

            def wait_gathers(s):
                for p in range(nparts):
                    pltpu.make_async_copy(tabs[p].at[ibufs[p].at[0]],
                                          rslots[s][p], gsems[s]).wait()

            def compute(jj, s):
                rbufs = rslots[s]

                def stage(g, _):
                    av_v = avb[jj, pl.ds(g * L, L)]
                    for i in range(L):
                        e = g * L + i
                        av = av_v[i]
                        cb[e, pl.ds(0, L)] = av * wl0 + b0
                        cb[e, pl.ds(L, L)] = av * wl1 + b1
                    return 0
                lax.fori_loop(0, CH // L, stage, 0)

                def edge(e, _):
                    x0 = rbufs[0][e, pl.ds(0, L)] + cb[e, pl.ds(0, L)]
                    x1 = rbufs[0][e, pl.ds(L, L)] + cb[e, pl.ds(L, L)]
                    for p in range(1, nparts):
                        x0 = x0 + rbufs[p][e, pl.ds(0, L)]
                        x1 = x1 + rbufs[p][e, pl.ds(L, L)]
                    msg[e, pl.ds(0, L)] = x0 / (1.0 + jnp.exp(-x0))
                    msg[e, pl.ds(L, L)] = x1 / (1.0 + jnp.exp(-x1))
                    return 0
                lax.fori_loop(0, CH, edge, 0)
                pltpu.sync_copy(msg, acc.at[ibufs[1].at[jj]], add=True)

            def super_body(js, _):
                row = tile_row0 + js * SUP
                descs = [
                    pltpu.async_copy(idx_arrs[p].at[pl.ds(row, SUP)],
                                     ibufs[p], si)
                    for p in range(nparts)
                ]
                descs.append(pltpu.async_copy(avsrc.at[pl.ds(row, SUP)],
                                              avb, si))
                for d in descs:
                    d.wait()

                if graph == 0:
                    fire(0, 0)
                    fire(1, 1)

                    def qbody(q, _):
                        k0 = 3 * q
                        for m in range(3):
                            wait_gathers(m)
                            compute(k0 + m, m)
                            fire(k0 + m + 2, (m + 2) % 3)
                        return 0
                    lax.fori_loop(0, SUP // 3 - 1, qbody, 0)

                    fire(SUP - 1, (SUP - 1) % 3)
                    for m in range(3):
                        wait_gathers(m)
                        compute(SUP - 3 + m, m)
                else:
                    fire(0, 0)

                    def qbody(q, _):
                        k0 = 2 * q
                        fire(k0 + 1, 1)
                        wait_gathers(0)
                        compute(k0, 0)
                        fire(k0 + 2, 0)
                        wait_gathers(1)
                        compute(k0 + 1, 1)
                        return 0
                    lax.fori_loop(0, SUP // 2 - 1, qbody, 0)

                    fire(SUP - 1, 1)
                    wait_gathers(0)
                    compute(SUP - 2, 0)
                    wait_gathers(1)
                    compute(SUP - 1, 1)
                return 0
            lax.fori_loop(0, nsup, super_body, 0)

            plsc.subcore_barrier()
            pltpu.sync_copy(acc.at[pl.ds(rows0, ROWS_PER_TILE)],
                            out.at[s_dyn, pl.ds(rows0, ROWS_PER_TILE)])
            plsc.subcore_barrier()
            return 0
        lax.fori_loop(0, 2, round_body, 0)


def _node_update(h, agga, aggd, whh, wha, whd, bh):
    bn = 400

    def body(h_ref, a_ref, d_ref, whh_ref, wha_ref, whd_ref, bh_ref, o_ref):
        hb = h_ref[...]
        acc = jnp.dot(hb, whh_ref[...], preferred_element_type=jnp.float32)
        for s in range(NSL):
            acc = acc + jnp.dot(a_ref[s], wha_ref[s],
                                preferred_element_type=jnp.float32)
            acc = acc + jnp.dot(d_ref[s], whd_ref[s],
                                preferred_element_type=jnp.float32)
        o_ref[...] = hb + acc + bh_ref[...]

    return pl.pallas_call(
        body,
        grid=(N // bn,),
        in_specs=[
            pl.BlockSpec((bn, DIM), lambda i: (i, 0)),
            pl.BlockSpec((NSL, bn, SL), lambda i: (0, i, 0)),
            pl.BlockSpec((NSL, bn, SL), lambda i: (0, i, 0)),
            pl.BlockSpec((DIM, DIM), lambda i: (0, 0)),
            pl.BlockSpec((NSL, SL, DIM), lambda i: (0, 0, 0)),
            pl.BlockSpec((NSL, SL, DIM), lambda i: (0, 0, 0)),
            pl.BlockSpec((1, DIM), lambda i: (0, 0)),
        ],
        out_specs=pl.BlockSpec((bn, DIM), lambda i: (i, 0)),
        out_shape=jax.ShapeDtypeStruct((N, DIM), jnp.float32),
    )(h, agga, aggd, whh, wha, whd, bh)


def _pad_edges(idx, val, e_pad):
    e = idx.shape[0]
    pad = e_pad - e
    cols = tuple(
        jnp.concatenate([idx[:, k], jnp.full((pad,), N, jnp.int32)])
        .reshape(-1, CH)
        for k in range(idx.shape[1]))
    v = jnp.concatenate([val[:, 0], jnp.zeros((pad,), jnp.float32)]).reshape(-1, CH)
    return cols, v


def kernel(h, a_idx, a_val, d_idx, d_val, W_a, b_a, W_d, b_d, W_h, b_h):
    wa_parts = W_a[:3 * DIM].reshape(3, DIM, DIM)
    wla = W_a[3 * DIM]
    ba = b_a
    wd_parts = W_d[:4 * DIM].reshape(4, DIM, DIM)
    wld = W_d[4 * DIM]
    bd = b_d
    whh = W_h[:DIM]
    wha = W_h[DIM:2 * DIM].reshape(NSL, SL, DIM)
    whd = W_h[2 * DIM:3 * DIM].reshape(NSL, SL, DIM)
    bh = b_h.reshape(1, DIM)

    h_pad = jnp.pad(h, ((0, NPT - N), (0, 0)))
    ta, td = _precompute_tables(h_pad, wa_parts, wd_parts)

    (i0a, i1a, i2a), ava = _pad_edges(a_idx, a_val, A_PAD)
    (i0d, i1d, i2d, i3d), avd = _pad_edges(d_idx, d_val, D_PAD)

    agga, aggd = _sc_messages(ta, td, i0a, i1a, i2a, ava,
                              i0d, i1d, i2d, i3d, avd, wla, ba, wld, bd)

    return _node_update(h, agga, aggd, whh, wha, whd, bh)
